# baseline (speedup 1.0000x reference)
import jax
import jax.numpy as jnp
from jax import lax
from jax.experimental import pallas as pl
from jax.experimental.pallas import tpu as pltpu
from jax.experimental.pallas import tpu_sc as plsc

B, S, F, V, D = 1024, 200, 26, 1000, 128
MAX_WAVELENGTH = 10000.0

NC, NS, L = 2, 16, 16
NW = NC * NS
TOKENS = B * S
TPW = TOKENS // NW
W = D // 4
C = 800
NCHUNK = TPW // C
NG = C // L
OSTG = 80
BIAS = 128 * F
MASK = 0x00FF00FF


def _pos_encoding():
    position = jnp.arange(S, dtype=jnp.float32)
    min_freq = jnp.float32(1.0 / MAX_WAVELENGTH)
    timescales = jnp.power(
        min_freq, (2 * (jnp.arange(D) // 2)).astype(jnp.float32) / jnp.float32(D)
    )
    angles = position[:, None] * timescales[None, :]
    cos_mask = (jnp.arange(D) % 2).astype(jnp.float32)
    return jnp.sin(angles) * (1.0 - cos_mask) + jnp.cos(angles) * cos_mask


def _unpack_bf16(word, lo):
    if lo:
        return lax.bitcast_convert_type(lax.shift_left(word, 16), jnp.float32)
    return lax.bitcast_convert_type(
        lax.bitwise_and(word, jnp.int32(-65536)), jnp.float32
    )


def _body(tab_hbm, x_hbm, pos_hbm, scl_hbm, out_hbm,
          tabf, acc_a, acc_b, x_v, pos_v, scl_v, out_v, sem0):
    wid = lax.axis_index("s") * NC + lax.axis_index("c")
    tok0 = wid * TPW

    pltpu.sync_copy(pos_hbm, pos_v)
    pltpu.sync_copy(scl_hbm, scl_v)

    def chunk(c, _):
        tbase = tok0 + c * C
        pltpu.sync_copy(x_hbm.at[pl.ds(tbase * F, C * F)], x_v)
        iota = lax.iota(jnp.int32, L)

        def zgroup(g, _):
            zero = iota - iota
            for w in range(W):
                acc_a[g, w, pl.ds(0, L)] = zero
                acc_b[g, w, pl.ds(0, L)] = zero
            return ()

        lax.fori_loop(0, NG, zgroup, (), unroll=False)

        def field_pass(f):
            tl = tabf

            def group(g, _):
                xpos = (g * (L * F) + f) + iota * F
                rowid = plsc.load_gather(x_v, [xpos])
                rbase = rowid * W
                sl = pl.ds(0, L)
                for w in range(W):
                    wv = plsc.load_gather(tl, [rbase + w])
                    pa = lax.bitwise_and(wv, jnp.int32(MASK))
                    pb = lax.bitwise_and(
                        lax.shift_right_logical(wv, 8), jnp.int32(MASK)
                    )
                    plsc.addupdate(acc_a.at[g, w, sl], pa)
                    plsc.addupdate(acc_b.at[g, w, sl], pb)
                return ()

            lax.fori_loop(0, NG, group, (), unroll=False)

        def fstep(f, _):
            pltpu.sync_copy(tab_hbm.at[pl.ds(f * V * W, V * W)], tabf)
            field_pass(f)
            return ()

        lax.fori_loop(0, F, fstep, (), unroll=False)

        scale = scl_v[pl.ds(0, L)]

        def fgroup(g, _):
            gb = g * L
            srow = lax.rem(gb + iota, S)
            pbase = srow * (D // 2)
            outbase = (lax.rem(gb, OSTG) + iota) * D
            sl = pl.ds(0, L)
            for w in range(W):
                wa = acc_a[g, w, sl]
                wb = acc_b[g, w, sl]
                pw0 = plsc.load_gather(pos_v, [pbase + 2 * w])
                pw1 = plsc.load_gather(pos_v, [pbase + 2 * w + 1])
                cols = (
                    (4 * w + 0, lax.bitwise_and(wa, jnp.int32(0xFFFF)), pw0, 1),
                    (4 * w + 1, lax.shift_right_logical(wa, 16), pw0, 0),
                    (4 * w + 2, lax.bitwise_and(wb, jnp.int32(0xFFFF)), pw1, 1),
                    (4 * w + 3, lax.shift_right_logical(wb, 16), pw1, 0),
                )
                for col, half, pw, lo in cols:
                    val = (
                        (half - jnp.int32(BIAS)).astype(jnp.float32) * scale
                        + _unpack_bf16(pw, lo)
                    )
                    plsc.store_scatter(out_v, [outbase + col], val)

            @pl.when(lax.rem(gb + L, OSTG) == 0)
            def _():
                pltpu.sync_copy(
                    out_v,
                    out_hbm.at[pl.ds((tbase + gb + L - OSTG) * D, OSTG * D)],
                )
            return ()

        lax.fori_loop(0, NG, fgroup, (), unroll=False)
        return ()

    lax.fori_loop(0, NCHUNK, chunk, (), unroll=False)


@jax.jit
def kernel(x, tables):
    x_flat = x.reshape(-1)
    scale = jnp.max(jnp.abs(tables)) / jnp.float32(127.0)
    q = jnp.round(tables.reshape(F * V, D) / scale).astype(jnp.int32) + 128
    perm = []
    for w in range(W):
        for b in (0, 2, 1, 3):
            perm.append(4 * w + b)
    tab8 = q.astype(jnp.uint8)[:, jnp.array(perm)]
    tab_flat = lax.bitcast_convert_type(
        tab8.reshape(F * V, W, 4), jnp.int32
    ).reshape(-1)

    pos_bf = _pos_encoding().astype(jnp.bfloat16)
    pos_pk = lax.bitcast_convert_type(
        pos_bf.reshape(S, D // 2, 2), jnp.int32
    ).reshape(-1)

    scl = jnp.full((L,), scale, jnp.float32)

    mesh = plsc.VectorSubcoreMesh(core_axis_name="c", subcore_axis_name="s",
                                  num_cores=NC, num_subcores=NS)
    run = pl.kernel(
        _body,
        out_type=jax.ShapeDtypeStruct((TOKENS * D,), jnp.float32),
        mesh=mesh,
        compiler_params=pltpu.CompilerParams(use_tc_tiling_on_sc=False, needs_layout_passes=False),
        scratch_types=[
            pltpu.VMEM((V * W,), jnp.int32),
            pltpu.VMEM((NG, W, L), jnp.int32),
            pltpu.VMEM((NG, W, L), jnp.int32),
            pltpu.VMEM((C * F,), jnp.int32),
            pltpu.VMEM((S * D // 2,), jnp.int32),
            pltpu.VMEM((L,), jnp.float32),
            pltpu.VMEM((OSTG * D,), jnp.float32),
            pltpu.SemaphoreType.DMA,
        ],
    )
    out = run(tab_flat, x_flat, pos_pk, scl)
    return out.reshape(B, S, D)


# speedup vs baseline: 2.0290x; 2.0290x over previous
import jax
import jax.numpy as jnp
from jax import lax
from jax.experimental import pallas as pl
from jax.experimental.pallas import tpu as pltpu
from jax.experimental.pallas import tpu_sc as plsc

B, S, F, V, D = 1024, 200, 26, 1000, 128
MAX_WAVELENGTH = 10000.0

NC, NS, L = 2, 16, 16
NW = NC * NS
TOKENS = B * S
TPW = TOKENS // NW
TB = 8
NBLK = TPW // TB
BLK_IDX = TB * F
SBLK = S // TB
W = D // 4
NBUF = 2


def _pos_encoding():
    position = jnp.arange(S, dtype=jnp.float32)
    min_freq = jnp.float32(1.0 / MAX_WAVELENGTH)
    timescales = jnp.power(
        min_freq, (2 * (jnp.arange(D) // 2)).astype(jnp.float32) / jnp.float32(D)
    )
    angles = position[:, None] * timescales[None, :]
    cos_mask = (jnp.arange(D) % 2).astype(jnp.float32)
    return jnp.sin(angles) * (1.0 - cos_mask) + jnp.cos(angles) * cos_mask


def _body(tab_hbm, x_hbm, offs_hbm, pos_hbm, scl_hbm, out_hbm,
          pos_v, offs_v, scl_v, x_v0, x_v1, idx_v0, idx_v1,
          rows_v0, rows_v1, out_v, sem0, sem1):
    wid = lax.axis_index("s") * NC + lax.axis_index("c")
    tok0 = wid * TPW
    sems = (sem0, sem1)
    x_bufs = (x_v0, x_v1)
    idx_bufs = (idx_v0, idx_v1)
    row_bufs = (rows_v0, rows_v1)

    pltpu.sync_copy(pos_hbm, pos_v)
    pltpu.sync_copy(offs_hbm, offs_v)
    pltpu.sync_copy(scl_hbm, scl_v)

    def start(blk, buf):
        base = (tok0 + blk * TB) * F
        pltpu.sync_copy(x_hbm.at[pl.ds(base, BLK_IDX)], x_bufs[buf])
        for i in range(BLK_IDX // L):
            sl = pl.ds(i * L, L)
            idx_bufs[buf][sl] = x_bufs[buf][sl] + offs_v[sl]
        pltpu.async_copy(tab_hbm.at[idx_bufs[buf]], row_bufs[buf], sems[buf])

    def finish(blk, buf):
        pltpu.make_async_copy(
            tab_hbm.at[idx_bufs[buf]], row_bufs[buf], sems[buf]
        ).wait()
        s0 = lax.rem(blk, SBLK) * TB
        scale = scl_v[pl.ds(0, L)]
        for t in range(TB):
            srow = s0 + t
            for k in range(W // L):
                accs = [None] * 4
                for f in range(F):
                    w = row_bufs[buf][t * F + f, pl.ds(L * k, L)]
                    vals = (
                        lax.shift_right_arithmetic(lax.shift_left(w, 24), 24),
                        lax.shift_right_arithmetic(lax.shift_left(w, 16), 24),
                        lax.shift_right_arithmetic(lax.shift_left(w, 8), 24),
                        lax.shift_right_arithmetic(w, 24),
                    )
                    for b in range(4):
                        accs[b] = vals[b] if accs[b] is None else accs[b] + vals[b]
                for b in range(4):
                    d0 = 4 * L * k + L * b
                    out_v[t, pl.ds(d0, L)] = (
                        accs[b].astype(jnp.float32) * scale
                        + pos_v[srow, pl.ds(d0, L)]
                    )
        pltpu.sync_copy(out_v, out_hbm.at[pl.ds(tok0 + blk * TB, TB)])

    for i in range(NBUF - 1):
        start(i, i)

    def group(gq, _):
        base = gq * NBUF
        for i in range(NBUF):
            blk = base + i

            @pl.when(blk + NBUF - 1 < NBLK)
            def _():
                start(blk + NBUF - 1, (i + NBUF - 1) % NBUF)

            finish(blk, i)
        return ()

    lax.fori_loop(0, NBLK // NBUF, group, (), unroll=False)


@jax.jit
def kernel(x, tables):
    x_flat = x.reshape(-1)
    scale = jnp.max(jnp.abs(tables)) / jnp.float32(127.0)
    q = jnp.round(tables.reshape(F * V, D) / scale).astype(jnp.int8)
    cols = []
    for p in range(D):
        jj, b = p // 4, p % 4
        cols.append(64 * (jj // L) + 16 * b + (jj % L))
    tab8 = q[:, jnp.array(cols)]
    tab_flat = lax.bitcast_convert_type(tab8.reshape(F * V, W, 4), jnp.int32)
    offs = (jnp.arange(BLK_IDX, dtype=jnp.int32) % F) * V
    pos = _pos_encoding()
    scl = jnp.full((L,), scale, jnp.float32)

    mesh = plsc.VectorSubcoreMesh(core_axis_name="c", subcore_axis_name="s",
                                  num_cores=NC, num_subcores=NS)
    run = pl.kernel(
        _body,
        out_type=jax.ShapeDtypeStruct((TOKENS, D), jnp.float32),
        mesh=mesh,
        compiler_params=pltpu.CompilerParams(use_tc_tiling_on_sc=False),
        scratch_types=[
            pltpu.VMEM((S, D), jnp.float32),
            pltpu.VMEM((BLK_IDX,), jnp.int32),
            pltpu.VMEM((L,), jnp.float32),
        ] + [pltpu.VMEM((BLK_IDX,), jnp.int32) for _ in range(NBUF)]
          + [pltpu.VMEM((BLK_IDX,), jnp.int32) for _ in range(NBUF)]
          + [pltpu.VMEM((BLK_IDX, W), jnp.int32) for _ in range(NBUF)]
          + [pltpu.VMEM((TB, D), jnp.float32)]
          + [pltpu.SemaphoreType.DMA for _ in range(NBUF)],
    )
    out = run(tab_flat, x_flat, offs, pos, scl)
    return out.reshape(B, S, D)
